# baseline (speedup 1.0000x reference)
import jax
import jax.numpy as jnp
from jax.experimental import pallas as pl
from jax.experimental.pallas import tpu as pltpu

_BF16 = jnp.bfloat16
_F32 = jnp.float32

_STAGES = (1, 2, 2, 2)


def _conv1_7x7(x4, w_ref, t_ref):
    n = x4.shape[0]
    xp = jnp.pad(x4, ((0, 0), (2, 2), (8, 8)))
    gs = []
    for i in range(7):
        for j in range(7):
            g = jax.lax.slice(xp, (0, i, 4 * j), (n, i + 14, 4 * j + 56))
            gs.append(g.reshape(n * 14, 56))
    gcat = jnp.concatenate(gs, axis=1)
    y = jnp.dot(gcat, w_ref[...], preferred_element_type=_F32) + t_ref[...]
    return jnp.maximum(y, 0.0)


def _maxpool3_merged(y, n):
    y3 = y.reshape(n, 14, 896)
    yb = jnp.pad(y3.astype(_BF16), ((0, 0), (1, 1), (64, 64)))
    m = y3
    for di in range(3):
        for dj in range(3):
            if di == 1 and dj == 1:
                continue
            v = jax.lax.slice(yb, (0, di, 64 * dj),
                              (n, di + 14, 64 * dj + 896))
            m = jnp.maximum(m, v.astype(_F32))
    return m


def _unmerge_lanes(y, n, hw, c):
    rows = y.shape[0]
    pieces = []
    for k in range(hw):
        p = jax.lax.slice(y, (0, k * c), (rows, (k + 1) * c))
        pieces.append(p.reshape(rows, 1, c))
    return jnp.concatenate(pieces, axis=1).reshape(n, hw, hw, c)


def _gemm(a, w_ref, t_ref, relu):
    y = jnp.dot(a.astype(_BF16), w_ref[...],
                preferred_element_type=_F32) + t_ref[...]
    return jnp.maximum(y, 0.0) if relu else y


def _maxpool3(a4):
    n, h, w, c = a4.shape
    ab = jnp.pad(a4.astype(_BF16), ((0, 0), (1, 1), (1, 1), (0, 0)))
    m = a4
    for di in range(3):
        for dj in range(3):
            if di == 1 and dj == 1:
                continue
            v = jax.lax.slice(ab, (0, di, dj, 0), (n, di + h, dj + w, c))
            m = jnp.maximum(m, v.astype(_F32))
    return m


def _stride2_slice(x, di, dj, oh, ow):
    n, h, w, c = x.shape
    x = x.reshape(n, h // 2, 2, w, c)
    x = jax.lax.slice(x, (0, di // 2, di % 2, 0, 0),
                      (n, di // 2 + oh, di % 2 + 1, w, c))
    x = x.reshape(n, oh, w // 2, 2, c)
    x = jax.lax.slice(x, (0, 0, dj // 2, dj % 2, 0),
                      (n, oh, dj // 2 + ow, dj % 2 + 1, c))
    return x.reshape(n, oh, ow, c)


def _conv3x3(a4, w_ref, t_ref, stride):
    n, h, w, cin = a4.shape
    oh = (h - 1) // stride + 1
    ow = (w - 1) // stride + 1
    phi = 1 + ((h + 2) % 2 if stride == 2 else 0)
    pwi = 1 + ((w + 2) % 2 if stride == 2 else 0)
    ab = jnp.pad(a4.astype(_BF16), ((0, 0), (1, phi), (1, pwi), (0, 0)))
    gs = []
    for di in range(3):
        for dj in range(3):
            if stride == 1:
                g = jax.lax.slice(ab, (0, di, dj, 0),
                                  (n, di + h, dj + w, cin))
            else:
                g = _stride2_slice(ab, di, dj, oh, ow)
            gs.append(g.reshape(n * oh * ow, cin))
    y = jnp.dot(jnp.concatenate(gs, axis=1), w_ref[...],
                preferred_element_type=_F32) + t_ref[...]
    return jnp.maximum(y, 0.0)


def _bottleneck(a4, w1, t1, w2, t2, w3, t3, wd, td, stride):
    n, h, w, cin = a4.shape
    a = a4.reshape(n * h * w, cin)
    o1 = _gemm(a, w1, t1, relu=True)
    inter = o1.shape[1]
    o2 = _conv3x3(o1.reshape(n, h, w, inter), w2, t2, stride)
    o3 = _gemm(o2, w3, t3, relu=False)
    oh2 = (h - 1) // stride + 1
    ow2 = (w - 1) // stride + 1
    if stride != 1:
        xs = a4
        if h % 2:
            xs = jnp.pad(xs, ((0, 0), (0, 1), (0, 0), (0, 0)))
        if w % 2:
            xs = jnp.pad(xs, ((0, 0), (0, 0), (0, 1), (0, 0)))
        xi = _stride2_slice(xs, 0, 0, oh2, ow2).reshape(-1, cin)
    else:
        xi = a
    idn = _gemm(xi, wd, td, relu=False)
    return jnp.maximum(o3 + idn, 0.0).reshape(n, oh2, ow2, o3.shape[1])


def _reduce_conv(a4, w_ref, t_ref):
    n, h, w, cin = a4.shape
    cout = w_ref.shape[1]
    xb = a4.reshape(n * h * w, cin).astype(_BF16)
    acc = jnp.zeros((n, 4, 4, cout), _F32)
    for di in range(3):
        for dj in range(3):
            t = di * 3 + dj
            yt = jnp.dot(xb, w_ref[t * cin:(t + 1) * cin, :],
                         preferred_element_type=_F32)
            ytp = jnp.pad(yt.reshape(n, h, w, cout),
                          ((0, 0), (2, 2), (2, 2), (0, 0)))
            acc = acc + jax.lax.slice(ytp, (0, di, dj, 0),
                                      (n, di + 4, dj + 4, cout))
    return acc.reshape(n * 16, cout) + t_ref[...]


_TRUNC = None


def _make_body(n_blk):
    def body(*refs):
        o_ref = refs[-1]
        a = _conv1_7x7(refs[0][...], refs[1], refs[2])
        m = _maxpool3_merged(a, n_blk)
        mb = m.astype(_BF16).reshape(n_blk * 14, 896)
        a4 = _unmerge_lanes(mb, n_blk, 14, 64)
        i = 3
        for si, stride in enumerate(_STAGES):
            if _TRUNC is not None and si >= _TRUNC:
                break
            w1, t1, w2, t2, w3, t3, wd, td = refs[i:i + 8]
            i += 8
            a4 = _bottleneck(a4, w1, t1, w2, t2, w3, t3, wd, td, stride)
        if _TRUNC is None:
            o_ref[...] = _reduce_conv(a4, refs[-3], refs[-2])
        else:
            o_ref[...] = jnp.full((n_blk * 16, 512), jnp.sum(a4), _F32)
    return body


def _full_spec(shape):
    nd = len(shape)
    return pl.BlockSpec(shape, lambda i, _nd=nd: (0,) * _nd)


def kernel(x, conv1_w, conv1_shift,
           l0_c1_w, l0_c1_shift, l0_c2_w, l0_c2_shift,
           l0_c3_w, l0_c3_shift, l0_down_w, l0_down_shift,
           l1_c1_w, l1_c1_shift, l1_c2_w, l1_c2_shift,
           l1_c3_w, l1_c3_shift, l1_down_w, l1_down_shift,
           l2_c1_w, l2_c1_shift, l2_c2_w, l2_c2_shift,
           l2_c3_w, l2_c3_shift, l2_down_w, l2_down_shift,
           l3_c1_w, l3_c1_shift, l3_c2_w, l3_c2_shift,
           l3_c3_w, l3_c3_shift, l3_down_w, l3_down_shift,
           reduce_w, reduce_shift):
    n = x.shape[0]
    ncores = 1
    n_blk = n // ncores

    xh = jnp.transpose(x, (0, 2, 3, 1)).astype(_BF16).reshape(n, 16, 64)

    w49 = conv1_w.reshape(49, 4, 64)
    eye14 = jnp.eye(14, dtype=conv1_w.dtype)
    c1bd = (w49[:, None, :, None, :]
            * eye14[None, :, None, :, None]).reshape(49 * 56, 14 * 64)
    c1bias = jnp.tile(conv1_shift, (1, 14))

    layer_args = []
    for c1w, c1s, c2w, c2s, c3w, c3s, dw, ds in (
            (l0_c1_w, l0_c1_shift, l0_c2_w, l0_c2_shift,
             l0_c3_w, l0_c3_shift, l0_down_w, l0_down_shift),
            (l1_c1_w, l1_c1_shift, l1_c2_w, l1_c2_shift,
             l1_c3_w, l1_c3_shift, l1_down_w, l1_down_shift),
            (l2_c1_w, l2_c1_shift, l2_c2_w, l2_c2_shift,
             l2_c3_w, l2_c3_shift, l2_down_w, l2_down_shift),
            (l3_c1_w, l3_c1_shift, l3_c2_w, l3_c2_shift,
             l3_c3_w, l3_c3_shift, l3_down_w, l3_down_shift)):
        layer_args.extend([c1w, c1s, c2w, c2s, c3w, c3s, dw, ds])

    args = ([xh, c1bd, c1bias] + layer_args
            + [reduce_w, reduce_shift])

    out_rows_per_blk = n_blk * 16

    in_specs = [pl.BlockSpec((n_blk, 16, 64), lambda i: (i, 0, 0))]
    in_specs += [_full_spec(a.shape) for a in args[1:]]

    nbytes = sum(int(a.size) * a.dtype.itemsize for a in args)
    flops = 2 * (784 * 196 * 64
                 + 784 * 64 * 64 + 9 * 784 * 64 * 64
                 + 784 * 64 * 256 + 784 * 64 * 256
                 + 784 * 256 * 128 + 9 * 196 * 128 * 128
                 + 196 * 128 * 512 + 196 * 256 * 512
                 + 196 * 512 * 256 + 9 * 64 * 256 * 256
                 + 64 * 256 * 1024 + 64 * 512 * 1024
                 + 64 * 1024 * 512 + 9 * 16 * 512 * 512
                 + 16 * 512 * 2048 + 16 * 1024 * 2048
                 + 9 * 16 * 2048 * 512)
    out = pl.pallas_call(
        _make_body(n_blk),
        out_shape=jax.ShapeDtypeStruct((n * 16, 512), _F32),
        grid=(ncores,),
        in_specs=in_specs,
        out_specs=pl.BlockSpec((out_rows_per_blk, 512), lambda i: (i, 0)),
        compiler_params=pltpu.CompilerParams(
            dimension_semantics=("parallel",),
            vmem_limit_bytes=int(min(nbytes + (20 << 20), 60 << 20))),
        cost_estimate=pl.CostEstimate(flops=int(flops), transcendentals=0,
                                      bytes_accessed=int(nbytes)),
    )(*args)

    y = out.reshape(n, 4, 4, 512)
    return jnp.transpose(y, (0, 3, 1, 2))


# speedup vs baseline: 1.1378x; 1.1378x over previous
import jax
import jax.numpy as jnp
from jax.experimental import pallas as pl
from jax.experimental.pallas import tpu as pltpu

_BF16 = jnp.bfloat16
_F32 = jnp.float32

_STAGES = (1, 2, 2, 2)


def _conv1_7x7(x4, w_ref, t_ref):
    n = x4.shape[0]
    xp = jnp.pad(x4, ((0, 0), (2, 2), (8, 8)))
    r = jax.lax.broadcasted_iota(jnp.int32, (56, 896), 0)
    l = jax.lax.broadcasted_iota(jnp.int32, (56, 896), 1)
    mask = (r // 4) == (l // 64)
    acc = None
    for i in range(7):
        for j in range(7):
            t = i * 7 + j
            g = jax.lax.slice(xp, (0, i, 4 * j), (n, i + 14, 4 * j + 56))
            g2 = g.reshape(n * 14, 56)
            wt = w_ref[4 * t:4 * t + 4, :]
            wrow = jnp.concatenate([wt] * 14, axis=0)
            wtile = jnp.concatenate([wrow] * 14, axis=1)
            bt = jnp.where(mask, wtile, jnp.zeros((), _BF16))
            part = jnp.dot(g2, bt, preferred_element_type=_F32)
            acc = part if acc is None else acc + part
    bias = jnp.concatenate([t_ref[...]] * 14, axis=1)
    return jnp.maximum(acc + bias, 0.0)


def _maxpool3_merged(y, n):
    y3 = y.reshape(n, 14, 896)
    yb = jnp.pad(y3.astype(_BF16), ((0, 0), (1, 1), (64, 64)))
    m = y3
    for di in range(3):
        for dj in range(3):
            if di == 1 and dj == 1:
                continue
            v = jax.lax.slice(yb, (0, di, 64 * dj),
                              (n, di + 14, 64 * dj + 896))
            m = jnp.maximum(m, v.astype(_F32))
    return m


def _unmerge_lanes(y, n, hw, c):
    rows = y.shape[0]
    pieces = []
    for k in range(hw):
        p = jax.lax.slice(y, (0, k * c), (rows, (k + 1) * c))
        pieces.append(p.reshape(rows, 1, c))
    return jnp.concatenate(pieces, axis=1).reshape(n, hw, hw, c)


def _gemm(a, w_ref, t_ref, relu):
    y = jnp.dot(a.astype(_BF16), w_ref[...],
                preferred_element_type=_F32) + t_ref[...]
    return jnp.maximum(y, 0.0) if relu else y


def _stride2_slice(x, di, dj, oh, ow):
    n, h, w, c = x.shape
    x = x.reshape(n, h // 2, 2, w, c)
    x = jax.lax.slice(x, (0, di // 2, di % 2, 0, 0),
                      (n, di // 2 + oh, di % 2 + 1, w, c))
    x = x.reshape(n, oh, w // 2, 2, c)
    x = jax.lax.slice(x, (0, 0, dj // 2, dj % 2, 0),
                      (n, oh, dj // 2 + ow, dj % 2 + 1, c))
    return x.reshape(n, oh, ow, c)


def _conv3x3(a4, w_ref, t_ref, stride):
    n, h, w, cin = a4.shape
    oh = (h - 1) // stride + 1
    ow = (w - 1) // stride + 1
    phi = 1 + ((h + 2) % 2 if stride == 2 else 0)
    pwi = 1 + ((w + 2) % 2 if stride == 2 else 0)
    ab = jnp.pad(a4.astype(_BF16), ((0, 0), (1, phi), (1, pwi), (0, 0)))
    gs = []
    for di in range(3):
        for dj in range(3):
            if stride == 1:
                g = jax.lax.slice(ab, (0, di, dj, 0),
                                  (n, di + h, dj + w, cin))
            else:
                g = _stride2_slice(ab, di, dj, oh, ow)
            gs.append(g.reshape(n * oh * ow, cin))
    y = jnp.dot(jnp.concatenate(gs, axis=1), w_ref[...],
                preferred_element_type=_F32) + t_ref[...]
    return jnp.maximum(y, 0.0)


def _bottleneck(a4, w1, t1, w2, t2, w3, t3, wd, td, stride):
    n, h, w, cin = a4.shape
    a = a4.reshape(n * h * w, cin)
    o1 = _gemm(a, w1, t1, relu=True)
    inter = o1.shape[1]
    o2 = _conv3x3(o1.reshape(n, h, w, inter), w2, t2, stride)
    o3 = _gemm(o2, w3, t3, relu=False)
    oh2 = (h - 1) // stride + 1
    ow2 = (w - 1) // stride + 1
    if stride != 1:
        xs = a4
        if h % 2:
            xs = jnp.pad(xs, ((0, 0), (0, 1), (0, 0), (0, 0)))
        if w % 2:
            xs = jnp.pad(xs, ((0, 0), (0, 0), (0, 1), (0, 0)))
        xi = _stride2_slice(xs, 0, 0, oh2, ow2).reshape(-1, cin)
    else:
        xi = a
    idn = _gemm(xi, wd, td, relu=False)
    return jnp.maximum(o3 + idn, 0.0).reshape(n, oh2, ow2, o3.shape[1])


def _reduce_conv(a4, w_ref, t_ref):
    n, h, w, cin = a4.shape
    cout = w_ref.shape[1]
    xb = a4.reshape(n * h * w, cin).astype(_BF16)
    acc = jnp.zeros((n, 4, 4, cout), _F32)
    for di in range(3):
        for dj in range(3):
            t = di * 3 + dj
            yt = jnp.dot(xb, w_ref[t * cin:(t + 1) * cin, :],
                         preferred_element_type=_F32)
            ytp = jnp.pad(yt.reshape(n, h, w, cout),
                          ((0, 0), (2, 2), (2, 2), (0, 0)))
            acc = acc + jax.lax.slice(ytp, (0, di, dj, 0),
                                      (n, di + 4, dj + 4, cout))
    return acc.reshape(n * 16, cout) + t_ref[...]


def _make_body(n_blk):
    def body(*refs):
        o_ref = refs[-1]
        a = _conv1_7x7(refs[0][...], refs[1], refs[2])
        m = _maxpool3_merged(a, n_blk)
        mb = m.astype(_BF16).reshape(n_blk * 14, 896)
        a4 = _unmerge_lanes(mb, n_blk, 14, 64)
        i = 3
        for stride in _STAGES:
            w1, t1, w2, t2, w3, t3, wd, td = refs[i:i + 8]
            i += 8
            a4 = _bottleneck(a4, w1, t1, w2, t2, w3, t3, wd, td, stride)
        o_ref[...] = _reduce_conv(a4, refs[-3], refs[-2])
    return body


def _full_spec(shape):
    nd = len(shape)
    return pl.BlockSpec(shape, lambda i, _nd=nd: (0,) * _nd)


def kernel(x, conv1_w, conv1_shift,
           l0_c1_w, l0_c1_shift, l0_c2_w, l0_c2_shift,
           l0_c3_w, l0_c3_shift, l0_down_w, l0_down_shift,
           l1_c1_w, l1_c1_shift, l1_c2_w, l1_c2_shift,
           l1_c3_w, l1_c3_shift, l1_down_w, l1_down_shift,
           l2_c1_w, l2_c1_shift, l2_c2_w, l2_c2_shift,
           l2_c3_w, l2_c3_shift, l2_down_w, l2_down_shift,
           l3_c1_w, l3_c1_shift, l3_c2_w, l3_c2_shift,
           l3_c3_w, l3_c3_shift, l3_down_w, l3_down_shift,
           reduce_w, reduce_shift):
    n = x.shape[0]
    ncores = 1
    n_blk = n // ncores

    xh = jnp.transpose(x, (0, 2, 3, 1)).astype(_BF16).reshape(n, 16, 64)

    layer_args = []
    for c1w, c1s, c2w, c2s, c3w, c3s, dw, ds in (
            (l0_c1_w, l0_c1_shift, l0_c2_w, l0_c2_shift,
             l0_c3_w, l0_c3_shift, l0_down_w, l0_down_shift),
            (l1_c1_w, l1_c1_shift, l1_c2_w, l1_c2_shift,
             l1_c3_w, l1_c3_shift, l1_down_w, l1_down_shift),
            (l2_c1_w, l2_c1_shift, l2_c2_w, l2_c2_shift,
             l2_c3_w, l2_c3_shift, l2_down_w, l2_down_shift),
            (l3_c1_w, l3_c1_shift, l3_c2_w, l3_c2_shift,
             l3_c3_w, l3_c3_shift, l3_down_w, l3_down_shift)):
        layer_args.extend([c1w, c1s, c2w, c2s, c3w, c3s, dw, ds])

    args = ([xh, conv1_w, conv1_shift] + layer_args
            + [reduce_w, reduce_shift])

    out_rows_per_blk = n_blk * 16

    in_specs = [pl.BlockSpec((n_blk, 16, 64), lambda i: (i, 0, 0))]
    in_specs += [_full_spec(a.shape) for a in args[1:]]

    nbytes = sum(int(a.size) * a.dtype.itemsize for a in args)
    flops = 2 * (784 * 196 * 64
                 + 784 * 64 * 64 + 9 * 784 * 64 * 64
                 + 784 * 64 * 256 + 784 * 64 * 256
                 + 784 * 256 * 128 + 9 * 196 * 128 * 128
                 + 196 * 128 * 512 + 196 * 256 * 512
                 + 196 * 512 * 256 + 9 * 64 * 256 * 256
                 + 64 * 256 * 1024 + 64 * 512 * 1024
                 + 64 * 1024 * 512 + 9 * 16 * 512 * 512
                 + 16 * 512 * 2048 + 16 * 1024 * 2048
                 + 9 * 16 * 2048 * 512)
    out = pl.pallas_call(
        _make_body(n_blk),
        out_shape=jax.ShapeDtypeStruct((n * 16, 512), _F32),
        grid=(ncores,),
        in_specs=in_specs,
        out_specs=pl.BlockSpec((out_rows_per_blk, 512), lambda i: (i, 0)),
        compiler_params=pltpu.CompilerParams(
            dimension_semantics=("parallel",),
            vmem_limit_bytes=int(min(nbytes + (20 << 20), 60 << 20))),
        cost_estimate=pl.CostEstimate(flops=int(flops), transcendentals=0,
                                      bytes_accessed=int(nbytes)),
    )(*args)

    y = out.reshape(n, 4, 4, 512)
    return jnp.transpose(y, (0, 3, 1, 2))
